# baseline (speedup 1.0000x reference)
import jax
import jax.numpy as jnp
from jax.experimental import pallas as pl
from jax.experimental.pallas import tpu as pltpu

B, H, W = 16, 384, 384
IMGS = 4

_C = (0.9998878719025601, -0.49636774398802214, 0.304670863083119,
      -0.15602693973930298, 0.0410640708360418)


def _loss_kernel(label_ref, x_ref, p_ref, n_ref, out_ref, acc_ref, cnt_ref):
    g = pl.program_id(0)

    @pl.when(g == 0)
    def _init():
        acc_ref[0] = 0.0
        cnt_ref[0] = 0.0

    total = 0.0
    count = 0.0
    for j in range(IMGS):
        x = x_ref[j]
        p = p_ref[j]
        n = n_ref[j]
        m = jnp.logical_and(p > 0.5, n > 0.5).astype(jnp.float32)
        y = label_ref[g * IMGS + j].astype(jnp.float32)
        ax = jnp.abs(x)
        um = jnp.exp(-ax) * m
        gp = jnp.float32(_C[4])
        for cf in (_C[3], _C[2], _C[1], _C[0]):
            gp = gp * um + cf
        s_log = jnp.sum(um * gp)
        s_x = jnp.sum(x * m)
        s_max = (s_x + jnp.sum(ax * m)) * 0.5
        total += (1.0 + y) * (s_max + s_log) - (2.0 * y) * s_x
        count += jnp.sum(m)
    acc_ref[0] += total
    cnt_ref[0] += count

    @pl.when(g == pl.num_programs(0) - 1)
    def _fini():
        out_ref[0] = acc_ref[0] / cnt_ref[0]


def kernel(cancer_logits, prostate_mask, needle_mask, label, involvement):
    grid_spec = pltpu.PrefetchScalarGridSpec(
        num_scalar_prefetch=1,
        grid=(B // IMGS,),
        in_specs=[
            pl.BlockSpec((IMGS, H, W), lambda g, lbl: (g, 0, 0)),
            pl.BlockSpec((IMGS, H, W), lambda g, lbl: (g, 0, 0)),
            pl.BlockSpec((IMGS, H, W), lambda g, lbl: (g, 0, 0)),
        ],
        out_specs=pl.BlockSpec(memory_space=pltpu.SMEM),
        scratch_shapes=[
            pltpu.SMEM((1,), jnp.float32),
            pltpu.SMEM((1,), jnp.float32),
        ],
    )
    out = pl.pallas_call(
        _loss_kernel,
        grid_spec=grid_spec,
        out_shape=jax.ShapeDtypeStruct((1,), jnp.float32),
    )(label.astype(jnp.int32), cancer_logits.reshape(B, H, W),
      prostate_mask.reshape(B, H, W), needle_mask.reshape(B, H, W))
    return out[0]


# speedup vs baseline: 1.2004x; 1.2004x over previous
import jax
import jax.numpy as jnp
from jax.experimental import pallas as pl
from jax.experimental.pallas import tpu as pltpu

B, H, W = 16, 384, 384
IMGS = 4

_C = (0.9998878719025601, -0.49636774398802214, 0.304670863083119,
      -0.15602693973930298, 0.0410640708360418)


def _loss_kernel(label_ref, x_ref, p_ref, n_ref, out_ref, acc_ref, cnt_ref):
    g = pl.program_id(0)

    @pl.when(g == 0)
    def _init():
        acc_ref[0] = 0.0
        cnt_ref[0] = 0.0

    total = 0.0
    count = 0.0
    for j in range(IMGS):
        x = x_ref[j]
        p = p_ref[j]
        n = n_ref[j]
        m = jnp.logical_and(p > 0.5, n > 0.5).astype(jnp.float32)
        y = label_ref[g * IMGS + j].astype(jnp.float32)
        u = jnp.exp(-jnp.abs(x))
        t = 1.0 + u * m
        v = t
        for _ in range(6):
            half = v.shape[0] // 2
            v = v[:half] * v[half:]
        s_log = jnp.sum(jnp.log(v))
        s_max = jnp.sum(m * jnp.maximum(x, 0.0))
        s_x = jnp.sum(m * x)
        total += (1.0 + y) * (s_max + s_log) - (2.0 * y) * s_x
        count += jnp.sum(m)
    acc_ref[0] += total
    cnt_ref[0] += count

    @pl.when(g == pl.num_programs(0) - 1)
    def _fini():
        out_ref[0] = acc_ref[0] / cnt_ref[0]


def kernel(cancer_logits, prostate_mask, needle_mask, label, involvement):
    grid_spec = pltpu.PrefetchScalarGridSpec(
        num_scalar_prefetch=1,
        grid=(B // IMGS,),
        in_specs=[
            pl.BlockSpec((IMGS, H, W), lambda g, lbl: (g, 0, 0)),
            pl.BlockSpec((IMGS, H, W), lambda g, lbl: (g, 0, 0)),
            pl.BlockSpec((IMGS, H, W), lambda g, lbl: (g, 0, 0)),
        ],
        out_specs=pl.BlockSpec(memory_space=pltpu.SMEM),
        scratch_shapes=[
            pltpu.SMEM((1,), jnp.float32),
            pltpu.SMEM((1,), jnp.float32),
        ],
    )
    out = pl.pallas_call(
        _loss_kernel,
        grid_spec=grid_spec,
        out_shape=jax.ShapeDtypeStruct((1,), jnp.float32),
    )(label.astype(jnp.int32), cancer_logits.reshape(B, H, W),
      prostate_mask.reshape(B, H, W), needle_mask.reshape(B, H, W))
    return out[0]


# speedup vs baseline: 1.2077x; 1.0061x over previous
import jax
import jax.numpy as jnp
from jax.experimental import pallas as pl
from jax.experimental.pallas import tpu as pltpu

B, H, W = 16, 384, 384
IMGS = 2
NSTEP = B // (2 * IMGS)


def _img_loss(x, p, n, y):
    m = jnp.logical_and(p > 0.5, n > 0.5).astype(jnp.float32)
    u = jnp.exp(-jnp.abs(x))
    t = 1.0 + u * m
    v = t
    for _ in range(6):
        half = v.shape[0] // 2
        v = v[:half] * v[half:]
    s_log = jnp.sum(jnp.log(v))
    s_max = jnp.sum(m * jnp.maximum(x, 0.0))
    s_x = jnp.sum(m * x)
    return (1.0 + y) * (s_max + s_log) - (2.0 * y) * s_x, jnp.sum(m)


def _loss_kernel(label_ref, xa_ref, xb_ref, pa_ref, pb_ref, na_ref, nb_ref,
                 out_ref, acc_ref, cnt_ref):
    g = pl.program_id(0)

    @pl.when(g == 0)
    def _init():
        acc_ref[0] = 0.0
        cnt_ref[0] = 0.0

    total = 0.0
    count = 0.0
    for grp, (x_ref, p_ref, n_ref) in enumerate(
            ((xa_ref, pa_ref, na_ref), (xb_ref, pb_ref, nb_ref))):
        for j in range(IMGS):
            y = label_ref[grp * (B // 2) + g * IMGS + j].astype(jnp.float32)
            s, c = _img_loss(x_ref[j], p_ref[j], n_ref[j], y)
            total += s
            count += c
    acc_ref[0] += total
    cnt_ref[0] += count

    @pl.when(g == pl.num_programs(0) - 1)
    def _fini():
        out_ref[0] = acc_ref[0] / cnt_ref[0]


def kernel(cancer_logits, prostate_mask, needle_mask, label, involvement):
    x = cancer_logits.reshape(B, H, W)
    p = prostate_mask.reshape(B, H, W)
    n = needle_mask.reshape(B, H, W)
    front = pl.BlockSpec((IMGS, H, W), lambda g, lbl: (g, 0, 0))
    back = pl.BlockSpec((IMGS, H, W), lambda g, lbl: (g + NSTEP, 0, 0))
    grid_spec = pltpu.PrefetchScalarGridSpec(
        num_scalar_prefetch=1,
        grid=(NSTEP,),
        in_specs=[front, back, front, back, front, back],
        out_specs=pl.BlockSpec(memory_space=pltpu.SMEM),
        scratch_shapes=[
            pltpu.SMEM((1,), jnp.float32),
            pltpu.SMEM((1,), jnp.float32),
        ],
    )
    out = pl.pallas_call(
        _loss_kernel,
        grid_spec=grid_spec,
        out_shape=jax.ShapeDtypeStruct((1,), jnp.float32),
    )(label.astype(jnp.int32), x, x, p, p, n, n)
    return out[0]


# speedup vs baseline: 1.2115x; 1.0031x over previous
import jax
import jax.numpy as jnp
from jax.experimental import pallas as pl
from jax.experimental.pallas import tpu as pltpu

B, H, W = 16, 384, 384
IMGS = 4


def _loss_kernel(label_ref, x_ref, p_ref, n_ref, out_ref, acc_ref, cnt_ref):
    g = pl.program_id(0)

    @pl.when(g == 0)
    def _init():
        acc_ref[0] = 0.0
        cnt_ref[0] = 0.0

    total = 0.0
    count = 0.0
    for j in range(IMGS):
        x = x_ref[j]
        p = p_ref[j]
        n = n_ref[j]
        m = jnp.logical_and(p > 0.5, n > 0.5).astype(jnp.float32)
        y = label_ref[g * IMGS + j].astype(jnp.float32)
        u = jnp.exp(-jnp.abs(x))
        t = 1.0 + u * m
        v = t
        for _ in range(6):
            half = v.shape[0] // 2
            v = v[:half] * v[half:]
        s_log = jnp.sum(jnp.log(v))
        s_max = jnp.sum(m * jnp.maximum(x, 0.0))
        s_x = jnp.sum(m * x)
        total += (1.0 + y) * (s_max + s_log) - (2.0 * y) * s_x
        count += jnp.sum(m)
    acc_ref[0] += total
    cnt_ref[0] += count

    @pl.when(g == pl.num_programs(0) - 1)
    def _fini():
        out_ref[0] = acc_ref[0] / cnt_ref[0]


def kernel(cancer_logits, prostate_mask, needle_mask, label, involvement):
    x = cancer_logits.reshape(B, H, W)
    p = prostate_mask.reshape(B, H, W)
    n = needle_mask.reshape(B, H, W)
    grid_spec = pltpu.PrefetchScalarGridSpec(
        num_scalar_prefetch=1,
        grid=(B // IMGS,),
        in_specs=[
            pl.BlockSpec((IMGS, H, W), lambda g, lbl: (g, 0, 0)),
            pl.BlockSpec((IMGS, H, W), lambda g, lbl: (g, 0, 0)),
            pl.BlockSpec((IMGS, H, W), lambda g, lbl: (g, 0, 0)),
        ],
        out_specs=pl.BlockSpec(memory_space=pltpu.SMEM),
        scratch_shapes=[
            pltpu.SMEM((1,), jnp.float32),
            pltpu.SMEM((1,), jnp.float32),
        ],
    )
    out = pl.pallas_call(
        _loss_kernel,
        grid_spec=grid_spec,
        out_shape=jax.ShapeDtypeStruct((1,), jnp.float32),
    )(label.astype(jnp.int32), x, p, n)
    return out[0]


# speedup vs baseline: 1.2421x; 1.0253x over previous
import jax
import jax.numpy as jnp
from jax.experimental import pallas as pl
from jax.experimental.pallas import tpu as pltpu

B, H, W = 16, 384, 384
CI = 2
NCH = B // CI
DEPTH = 4
CR = CI * H


def _loss_kernel(label_ref, x_hbm, p_hbm, n_hbm, out_ref,
                 xb, pb, nb, acc_ref, cnt_ref, sems):

    def start(k):
        slot = k % DEPTH
        rows = pl.ds(k * CR, CR)
        pltpu.make_async_copy(x_hbm.at[rows, :], xb.at[slot], sems.at[slot, 0]).start()
        pltpu.make_async_copy(p_hbm.at[rows, :], pb.at[slot], sems.at[slot, 1]).start()
        pltpu.make_async_copy(n_hbm.at[rows, :], nb.at[slot], sems.at[slot, 2]).start()

    def wait(k):
        slot = k % DEPTH
        rows = pl.ds(k * CR, CR)
        pltpu.make_async_copy(x_hbm.at[rows, :], xb.at[slot], sems.at[slot, 0]).wait()
        pltpu.make_async_copy(p_hbm.at[rows, :], pb.at[slot], sems.at[slot, 1]).wait()
        pltpu.make_async_copy(n_hbm.at[rows, :], nb.at[slot], sems.at[slot, 2]).wait()

    for k in range(DEPTH):
        start(k)

    total = 0.0
    count = 0.0
    for k in range(NCH):
        slot = k % DEPTH
        wait(k)
        for j in range(CI):
            x = xb[slot, pl.ds(j * H, H), :]
            p = pb[slot, pl.ds(j * H, H), :]
            n = nb[slot, pl.ds(j * H, H), :]
            m = jnp.logical_and(p > 0.5, n > 0.5).astype(jnp.float32)
            y = label_ref[k * CI + j].astype(jnp.float32)
            u = jnp.exp(-jnp.abs(x))
            t = 1.0 + u * m
            v = t
            for _ in range(6):
                half = v.shape[0] // 2
                v = v[:half] * v[half:]
            total += ((1.0 + y) * (jnp.sum(m * jnp.maximum(x, 0.0))
                                   + jnp.sum(jnp.log(v)))
                      - (2.0 * y) * jnp.sum(m * x))
            count += jnp.sum(m)
        if k + DEPTH < NCH:
            start(k + DEPTH)
    acc_ref[0] = total
    cnt_ref[0] = count
    out_ref[0] = total / count


def kernel(cancer_logits, prostate_mask, needle_mask, label, involvement):
    x = cancer_logits.reshape(B * H, W)
    p = prostate_mask.reshape(B * H, W)
    n = needle_mask.reshape(B * H, W)
    grid_spec = pltpu.PrefetchScalarGridSpec(
        num_scalar_prefetch=1,
        grid=(1,),
        in_specs=[
            pl.BlockSpec(memory_space=pl.ANY),
            pl.BlockSpec(memory_space=pl.ANY),
            pl.BlockSpec(memory_space=pl.ANY),
        ],
        out_specs=pl.BlockSpec(memory_space=pltpu.SMEM),
        scratch_shapes=[
            pltpu.VMEM((DEPTH, CR, W), jnp.float32),
            pltpu.VMEM((DEPTH, CR, W), jnp.float32),
            pltpu.VMEM((DEPTH, CR, W), jnp.float32),
            pltpu.SMEM((1,), jnp.float32),
            pltpu.SMEM((1,), jnp.float32),
            pltpu.SemaphoreType.DMA((DEPTH, 3)),
        ],
    )
    out = pl.pallas_call(
        _loss_kernel,
        grid_spec=grid_spec,
        out_shape=jax.ShapeDtypeStruct((1,), jnp.float32),
    )(label.astype(jnp.int32), x, p, n)
    return out[0]
